# Initial kernel scaffold; baseline (speedup 1.0000x reference)
import functools

import jax
import jax.numpy as jnp
from jax import lax
from jax.experimental import pallas as pl
from jax.experimental.pallas import tpu as pltpu
from jax.experimental.pallas import tpu_sc as plsc

D_IN = 768
WIDTH = 16384
K = 16
B = 64
NBLK = 8
CBLK = WIDTH // NBLK


def _mp_tc_body(x_ref, wn_ref, chosen_ref, mcoeff_ref,
                res_ref, g_ref, idxv_ref, idxs_ref, sem):
    res_ref[...] = x_ref[...]
    for t in range(K):
        residual = res_ref[...]
        m = jnp.full((B, 1), -1.0, dtype=jnp.float32)
        am = jnp.zeros((B, 1), dtype=jnp.int32)
        for b in range(NBLK):
            wb = wn_ref[b * CBLK:(b + 1) * CBLK, :]
            sc = lax.dot_general(residual, wb, (((1,), (1,)), ((), ())),
                                 preferred_element_type=jnp.float32)
            sc = jnp.maximum(sc, 0.0)
            mb = jnp.max(sc, axis=1, keepdims=True)
            iota = lax.broadcasted_iota(jnp.int32, (B, CBLK), 1) + b * CBLK
            amb = jnp.min(jnp.where(sc == mb, iota, WIDTH),
                          axis=1, keepdims=True)
            take = mb > m
            am = jnp.where(take, amb, am)
            m = jnp.maximum(mb, m)
        active = m > 1e-8
        mc = jnp.where(active, m, 0.0)
        chosen_ref[:, t:t + 1] = am
        mcoeff_ref[:, t:t + 1] = mc
        idxv_ref[...] = am
        cp = pltpu.make_async_copy(idxv_ref, idxs_ref, sem)
        cp.start()
        cp.wait()

        def gather_body(i, _):
            c = idxs_ref[i, 0]
            g_ref[pl.ds(i, 1), :] = wn_ref[pl.ds(c, 1), :]
            return 0

        lax.fori_loop(0, B, gather_body, 0)
        res_ref[...] = res_ref[...] - m * g_ref[...]


def _mp_tc(x, wn):
    return pl.pallas_call(
        _mp_tc_body,
        out_shape=[
            jax.ShapeDtypeStruct((B, K), jnp.int32),
            jax.ShapeDtypeStruct((B, K), jnp.float32),
        ],
        scratch_shapes=[
            pltpu.VMEM((B, D_IN), jnp.float32),
            pltpu.VMEM((B, D_IN), jnp.float32),
            pltpu.VMEM((B, 1), jnp.int32),
            pltpu.SMEM((B, 1), jnp.int32),
            pltpu.SemaphoreType.DMA,
        ],
    )(x, wn)


_SC_MESH = plsc.VectorSubcoreMesh(core_axis_name="c", subcore_axis_name="s")


@functools.partial(
    pl.kernel,
    mesh=_SC_MESH,
    out_type=[
        jax.ShapeDtypeStruct((B, WIDTH), jnp.float32),
        jax.ShapeDtypeStruct((B, D_IN), jnp.float32),
    ],
    scratch_types=[
        pltpu.VMEM((2, K), jnp.int32),
        pltpu.VMEM((2, K), jnp.float32),
        pltpu.VMEM((K, D_IN), jnp.float32),
        pltpu.VMEM((K, D_IN), jnp.float32),
        pltpu.VMEM((2, WIDTH), jnp.float32),
        pltpu.VMEM((2, D_IN), jnp.float32),
        pltpu.VMEM((D_IN,), jnp.float32),
        pltpu.SemaphoreType.DMA,
        pltpu.SemaphoreType.DMA,
    ],
)
def _sc_scatter_decode(chosen_hbm, mcoeff_hbm, w_hbm, bias_hbm,
                       z_hbm, xhat_hbm,
                       idx_v, mc_v, rows0_v, rows1_v, z_v, acc_v, bias_v,
                       sem0, sem1):
    wid = lax.axis_index("s") * 2 + lax.axis_index("c")
    base = wid * 2
    pltpu.sync_copy(chosen_hbm.at[pl.ds(base, 2)], idx_v)
    pltpu.sync_copy(mcoeff_hbm.at[pl.ds(base, 2)], mc_v)
    pltpu.sync_copy(bias_hbm, bias_v)
    idx0 = idx_v[0]
    idx1 = idx_v[1]
    cp0 = pltpu.make_async_copy(w_hbm.at[idx0], rows0_v, sem0)
    cp0.start()
    cp1 = pltpu.make_async_copy(w_hbm.at[idx1], rows1_v, sem1)
    cp1.start()

    zeros16 = jnp.zeros((16,), jnp.float32)

    def zero_body(j, _):
        z_v[0, pl.ds(j * 16, 16)] = zeros16
        z_v[1, pl.ds(j * 16, 16)] = zeros16
        return 0

    lax.fori_loop(0, WIDTH // 16, zero_body, 0)

    lane = lax.iota(jnp.int32, 16)
    for r in range(2):
        row_ids = jnp.full((16,), r, jnp.int32)
        idxr = idx_v[r]
        mcr = mc_v[r]
        for t in range(K):
            plsc.addupdate_scatter(z_v, [row_ids, idxr], mcr,
                                   mask=lane == t)

    cp0.wait()
    cp1.wait()

    for r, rows_v in ((0, rows0_v), (1, rows1_v)):
        for j in range(D_IN // 16):
            acc_v[r, pl.ds(j * 16, 16)] = bias_v[pl.ds(j * 16, 16)]
        for t in range(K):
            s = mc_v[r, t]
            for j in range(D_IN // 16):
                acc_v[r, pl.ds(j * 16, 16)] = (
                    acc_v[r, pl.ds(j * 16, 16)]
                    + s * rows_v[t, pl.ds(j * 16, 16)])

    pltpu.sync_copy(z_v, z_hbm.at[pl.ds(base, 2)])
    pltpu.sync_copy(acc_v, xhat_hbm.at[pl.ds(base, 2)])


def kernel(x, W, decoder_bias):
    norms = jnp.clip(jnp.linalg.norm(W, axis=1, keepdims=True), 1e-12, None)
    wn = W / norms
    chosen_t, mcoeff_t = _mp_tc(x, wn)
    z, x_hat = _sc_scatter_decode(chosen_t, mcoeff_t, W, decoder_bias)
    return (z, x_hat)


# speedup vs baseline: 3.3587x; 3.3587x over previous
import functools

import jax
import jax.numpy as jnp
from jax import lax
from jax.experimental import pallas as pl
from jax.experimental.pallas import tpu as pltpu
from jax.experimental.pallas import tpu_sc as plsc

D_IN = 768
WIDTH = 16384
K = 16
B = 64
NBLK = 8
CBLK = WIDTH // NBLK


def _mp_tc_body(x_ref, wn_ref, chosen_ref, mcoeff_ref,
                res_ref, g_ref, idxv_ref, idxs_ref, sem):
    res_ref[...] = x_ref[...]
    for t in range(K):
        residual = res_ref[...]
        m = jnp.full((B, 1), -1.0, dtype=jnp.float32)
        am = jnp.zeros((B, 1), dtype=jnp.int32)
        for b in range(NBLK):
            wb = wn_ref[b * CBLK:(b + 1) * CBLK, :]
            sc = lax.dot_general(residual, wb, (((1,), (1,)), ((), ())),
                                 preferred_element_type=jnp.float32)
            sc = jnp.maximum(sc, 0.0)
            mb = jnp.max(sc, axis=1, keepdims=True)
            iota = lax.broadcasted_iota(jnp.int32, (B, CBLK), 1) + b * CBLK
            amb = jnp.min(jnp.where(sc == mb, iota, WIDTH),
                          axis=1, keepdims=True)
            take = mb > m
            am = jnp.where(take, amb, am)
            m = jnp.maximum(mb, m)
        active = m > 1e-8
        mc = jnp.where(active, m, 0.0)
        chosen_ref[:, t:t + 1] = am
        mcoeff_ref[:, t:t + 1] = mc
        idxv_ref[...] = am
        cp = pltpu.make_async_copy(idxv_ref, idxs_ref, sem)
        cp.start()
        cp.wait()

        def gather_body(i, _):
            c = idxs_ref[i, 0]
            g_ref[pl.ds(i, 1), :] = wn_ref[pl.ds(c, 1), :]
            return 0

        lax.fori_loop(0, B, gather_body, 0)
        res_ref[...] = res_ref[...] - m * g_ref[...]


def _mp_tc(x, wn):
    return pl.pallas_call(
        _mp_tc_body,
        out_shape=[
            jax.ShapeDtypeStruct((B, K), jnp.int32),
            jax.ShapeDtypeStruct((B, K), jnp.float32),
        ],
        scratch_shapes=[
            pltpu.VMEM((B, D_IN), jnp.float32),
            pltpu.VMEM((B, D_IN), jnp.float32),
            pltpu.VMEM((B, 1), jnp.int32),
            pltpu.SMEM((B, 1), jnp.int32),
            pltpu.SemaphoreType.DMA,
        ],
    )(x, wn)


@functools.cache
def _sc_scatter_decode_fn():
    mesh = plsc.VectorSubcoreMesh(core_axis_name="c", subcore_axis_name="s")
    return pl.kernel(
        _sc_body,
        mesh=mesh,
        out_type=[
            jax.ShapeDtypeStruct((B, WIDTH), jnp.float32),
            jax.ShapeDtypeStruct((B, D_IN), jnp.float32),
        ],
        scratch_types=[
            pltpu.VMEM((2, K), jnp.int32),
            pltpu.VMEM((2, K), jnp.float32),
            pltpu.VMEM((K, D_IN), jnp.float32),
            pltpu.VMEM((K, D_IN), jnp.float32),
            pltpu.VMEM((WIDTH,), jnp.float32),
            pltpu.VMEM((WIDTH,), jnp.float32),
            pltpu.VMEM((2, D_IN), jnp.float32),
            pltpu.VMEM((D_IN,), jnp.float32),
            pltpu.SemaphoreType.DMA,
            pltpu.SemaphoreType.DMA,
        ],
        compiler_params=pltpu.CompilerParams(needs_layout_passes=False),
    )


def _sc_body(chosen_hbm, mcoeff_hbm, w_hbm, bias_hbm,
                       z_hbm, xhat_hbm,
                       idx_v, mc_v, rows0_v, rows1_v, z0_v, z1_v, acc_v,
                       bias_v, sem0, sem1):
    wid = lax.axis_index("s") * 2 + lax.axis_index("c")
    base = wid * 2
    pltpu.sync_copy(chosen_hbm.at[pl.ds(base, 2)], idx_v)
    pltpu.sync_copy(mcoeff_hbm.at[pl.ds(base, 2)], mc_v)
    pltpu.sync_copy(bias_hbm, bias_v)
    idx0 = idx_v[0]
    idx1 = idx_v[1]
    cp0 = pltpu.make_async_copy(w_hbm.at[idx0], rows0_v, sem0)
    cp0.start()
    cp1 = pltpu.make_async_copy(w_hbm.at[idx1], rows1_v, sem1)
    cp1.start()

    zeros16 = jnp.zeros((16,), jnp.float32)

    def zero_body(j, _):
        z0_v[pl.ds(j * 16, 16)] = zeros16
        z1_v[pl.ds(j * 16, 16)] = zeros16
        return 0

    lax.fori_loop(0, WIDTH // 16, zero_body, 0)

    lane = lax.iota(jnp.int32, 16)
    for r, z_row in ((0, z0_v), (1, z1_v)):
        idxr = idx_v[r]
        mcr = mc_v[r]
        for t in range(K):
            plsc.addupdate_scatter(z_row, [idxr], mcr, mask=lane == t)

    cp0.wait()
    cp1.wait()

    for r, rows_v in ((0, rows0_v), (1, rows1_v)):
        mcr_vec = mc_v[r]
        for j in range(D_IN // 16):
            acc_v[r, pl.ds(j * 16, 16)] = bias_v[pl.ds(j * 16, 16)]
        for t in range(K):
            s = mcr_vec[t]
            for j in range(D_IN // 16):
                acc_v[r, pl.ds(j * 16, 16)] = (
                    acc_v[r, pl.ds(j * 16, 16)]
                    + s * rows_v[t, pl.ds(j * 16, 16)])

    pltpu.sync_copy(z0_v, z_hbm.at[base])
    pltpu.sync_copy(z1_v, z_hbm.at[base + 1])
    pltpu.sync_copy(acc_v, xhat_hbm.at[pl.ds(base, 2)])


def kernel(x, W, decoder_bias):
    norms = jnp.clip(jnp.linalg.norm(W, axis=1, keepdims=True), 1e-12, None)
    wn = W / norms
    chosen_t, mcoeff_t = _mp_tc(x, wn)
    z, x_hat = _sc_scatter_decode_fn()(chosen_t, mcoeff_t, W, decoder_bias)
    return (z, x_hat)


# speedup vs baseline: 3.6243x; 1.0791x over previous
import functools

import jax
import jax.numpy as jnp
from jax import lax
from jax.experimental import pallas as pl
from jax.experimental.pallas import tpu as pltpu
from jax.experimental.pallas import tpu_sc as plsc

D_IN = 768
WIDTH = 16384
K = 16
B = 64
NBLK = 8
CBLK = WIDTH // NBLK


NORM_BLOCKS = 16
RBLK = WIDTH // NORM_BLOCKS


def _mp_tc_body(x_ref, w_ref, chosen_ref, mcoeff_ref,
                wn_ref, res_ref, g_ref, idxv_ref, idxs_ref, sem):
    i = pl.program_id(0)

    @pl.when(i < NORM_BLOCKS)
    def _normalize():
        wb = w_ref[...]
        norms = jnp.sqrt(jnp.sum(wb * wb, axis=1, keepdims=True))
        norms = jnp.clip(norms, 1e-12, None)
        wn_ref[pl.ds(i * RBLK, RBLK), :] = wb / norms

    @pl.when(i == NORM_BLOCKS)
    def _mp_loop():
        _mp_steps(x_ref, chosen_ref, mcoeff_ref,
                  wn_ref, res_ref, g_ref, idxv_ref, idxs_ref, sem)


def _mp_steps(x_ref, chosen_ref, mcoeff_ref,
              wn_ref, res_ref, g_ref, idxv_ref, idxs_ref, sem):
    res_ref[...] = x_ref[...]
    for t in range(K):
        residual = res_ref[...]
        m = jnp.full((B, 1), -1.0, dtype=jnp.float32)
        am = jnp.zeros((B, 1), dtype=jnp.int32)
        for b in range(NBLK):
            wb = wn_ref[b * CBLK:(b + 1) * CBLK, :]
            sc = lax.dot_general(residual, wb, (((1,), (1,)), ((), ())),
                                 preferred_element_type=jnp.float32)
            sc = jnp.maximum(sc, 0.0)
            mb = jnp.max(sc, axis=1, keepdims=True)
            iota = lax.broadcasted_iota(jnp.int32, (B, CBLK), 1) + b * CBLK
            amb = jnp.min(jnp.where(sc == mb, iota, WIDTH),
                          axis=1, keepdims=True)
            take = mb > m
            am = jnp.where(take, amb, am)
            m = jnp.maximum(mb, m)
        active = m > 1e-8
        mc = jnp.where(active, m, 0.0)
        chosen_ref[:, t:t + 1] = am
        mcoeff_ref[:, t:t + 1] = mc
        idxv_ref[...] = am
        cp = pltpu.make_async_copy(idxv_ref, idxs_ref, sem)
        cp.start()
        cp.wait()

        def gather_body(i, _):
            c = idxs_ref[i, 0]
            g_ref[pl.ds(i, 1), :] = wn_ref[pl.ds(c, 1), :]
            return 0

        lax.fori_loop(0, B, gather_body, 0)
        res_ref[...] = res_ref[...] - m * g_ref[...]


def _mp_tc(x, w):
    return pl.pallas_call(
        _mp_tc_body,
        grid=(NORM_BLOCKS + 1,),
        in_specs=[
            pl.BlockSpec((B, D_IN), lambda i: (0, 0)),
            pl.BlockSpec((RBLK, D_IN),
                         lambda i: (jnp.minimum(i, NORM_BLOCKS - 1), 0)),
        ],
        out_specs=[
            pl.BlockSpec((B, K), lambda i: (0, 0)),
            pl.BlockSpec((B, K), lambda i: (0, 0)),
        ],
        out_shape=[
            jax.ShapeDtypeStruct((B, K), jnp.int32),
            jax.ShapeDtypeStruct((B, K), jnp.float32),
        ],
        scratch_shapes=[
            pltpu.VMEM((WIDTH, D_IN), jnp.float32),
            pltpu.VMEM((B, D_IN), jnp.float32),
            pltpu.VMEM((B, D_IN), jnp.float32),
            pltpu.VMEM((B, 1), jnp.int32),
            pltpu.SMEM((B, 1), jnp.int32),
            pltpu.SemaphoreType.DMA,
        ],
    )(x, w)


@functools.cache
def _sc_scatter_decode_fn():
    mesh = plsc.VectorSubcoreMesh(core_axis_name="c", subcore_axis_name="s")
    return pl.kernel(
        _sc_body,
        mesh=mesh,
        out_type=[
            jax.ShapeDtypeStruct((B, WIDTH), jnp.float32),
            jax.ShapeDtypeStruct((B, D_IN), jnp.float32),
        ],
        scratch_types=[
            pltpu.VMEM((2, K), jnp.int32),
            pltpu.VMEM((2, K), jnp.float32),
            pltpu.VMEM((K, D_IN), jnp.float32),
            pltpu.VMEM((K, D_IN), jnp.float32),
            pltpu.VMEM((WIDTH,), jnp.float32),
            pltpu.VMEM((WIDTH,), jnp.float32),
            pltpu.VMEM((2, D_IN), jnp.float32),
            pltpu.VMEM((D_IN,), jnp.float32),
            pltpu.SemaphoreType.DMA,
            pltpu.SemaphoreType.DMA,
        ],
        compiler_params=pltpu.CompilerParams(needs_layout_passes=False),
    )


def _sc_body(chosen_hbm, mcoeff_hbm, w_hbm, bias_hbm,
                       z_hbm, xhat_hbm,
                       idx_v, mc_v, rows0_v, rows1_v, z0_v, z1_v, acc_v,
                       bias_v, sem0, sem1):
    wid = lax.axis_index("s") * 2 + lax.axis_index("c")
    base = wid * 2
    pltpu.sync_copy(chosen_hbm.at[pl.ds(base, 2)], idx_v)
    pltpu.sync_copy(mcoeff_hbm.at[pl.ds(base, 2)], mc_v)
    pltpu.sync_copy(bias_hbm, bias_v)
    idx0 = idx_v[0]
    idx1 = idx_v[1]
    cp0 = pltpu.make_async_copy(w_hbm.at[idx0], rows0_v, sem0)
    cp0.start()
    cp1 = pltpu.make_async_copy(w_hbm.at[idx1], rows1_v, sem1)
    cp1.start()

    zeros16 = jnp.zeros((16,), jnp.float32)

    def zero_body(j, _):
        z0_v[pl.ds(j * 16, 16)] = zeros16
        z1_v[pl.ds(j * 16, 16)] = zeros16
        return 0

    lax.fori_loop(0, WIDTH // 16, zero_body, 0)

    lane = lax.iota(jnp.int32, 16)
    for r, z_row in ((0, z0_v), (1, z1_v)):
        idxr = idx_v[r]
        mcr = mc_v[r]
        for t in range(K):
            plsc.addupdate_scatter(z_row, [idxr], mcr, mask=lane == t)

    cp0.wait()
    cp1.wait()

    for r, rows_v in ((0, rows0_v), (1, rows1_v)):
        mcr_vec = mc_v[r]
        for j in range(D_IN // 16):
            acc_v[r, pl.ds(j * 16, 16)] = bias_v[pl.ds(j * 16, 16)]
        for t in range(K):
            s = mcr_vec[t]
            for j in range(D_IN // 16):
                acc_v[r, pl.ds(j * 16, 16)] = (
                    acc_v[r, pl.ds(j * 16, 16)]
                    + s * rows_v[t, pl.ds(j * 16, 16)])

    pltpu.sync_copy(z0_v, z_hbm.at[base])
    pltpu.sync_copy(z1_v, z_hbm.at[base + 1])
    pltpu.sync_copy(acc_v, xhat_hbm.at[pl.ds(base, 2)])


def kernel(x, W, decoder_bias):
    chosen_t, mcoeff_t = _mp_tc(x, W)
    z, x_hat = _sc_scatter_decode_fn()(chosen_t, mcoeff_t, W, decoder_bias)
    return (z, x_hat)
